# baseline (speedup 1.0000x reference)
import jax
import jax.numpy as jnp
from jax.experimental import pallas as pl
from jax.experimental.pallas import tpu as pltpu

_BM = 512


def _matmul_kernel(x_ref, w_ref, o_ref, wb_ref):
    @pl.when(pl.program_id(0) == 0)
    def _():
        wb_ref[...] = w_ref[...].astype(jnp.bfloat16)

    o_ref[...] = jnp.dot(x_ref[...].astype(jnp.bfloat16), wb_ref[...],
                         preferred_element_type=jnp.float32)


def kernel(x, W0, W1, W2, W3, AP_path_wb):
    M, K = x.shape
    N = W0.shape[1]
    return pl.pallas_call(
        _matmul_kernel,
        grid=(M // _BM,),
        in_specs=[
            pl.BlockSpec((_BM, K), lambda i: (i, 0)),
            pl.BlockSpec((K, N), lambda i: (0, 0)),
        ],
        out_specs=pl.BlockSpec((_BM, N), lambda i: (i, 0)),
        out_shape=jax.ShapeDtypeStruct((M, N), jnp.float32),
        scratch_shapes=[pltpu.VMEM((K, N), jnp.bfloat16)],
        compiler_params=pltpu.CompilerParams(
            dimension_semantics=("arbitrary",)),
    )(x, W0)


# speedup vs baseline: 1.0121x; 1.0121x over previous
import jax
import jax.numpy as jnp
from jax.experimental import pallas as pl
from jax.experimental.pallas import tpu as pltpu

_BM = 512
_NQ = 4


def _matmul_kernel(x_ref, w_hbm, o_ref, wf_ref, wb_ref, *sems):
    i = pl.program_id(0)
    nb = w_hbm.shape[1] // _NQ
    xb = x_ref[...].astype(jnp.bfloat16)

    @pl.when(i == 0)
    def _():
        copies = [
            pltpu.make_async_copy(
                w_hbm.at[:, q * nb:(q + 1) * nb],
                wf_ref.at[:, q * nb:(q + 1) * nb],
                sems[q])
            for q in range(_NQ)
        ]
        for cp in copies:
            cp.start()
        for q, cp in enumerate(copies):
            cp.wait()
            sl = pl.ds(q * nb, nb)
            wb_ref[:, sl] = wf_ref[:, sl].astype(jnp.bfloat16)
            o_ref[:, sl] = jnp.dot(xb, wb_ref[:, sl],
                                   preferred_element_type=jnp.float32)

    @pl.when(i > 0)
    def _():
        o_ref[...] = jnp.dot(xb, wb_ref[...],
                             preferred_element_type=jnp.float32)


def kernel(x, W0, W1, W2, W3, AP_path_wb):
    M, K = x.shape
    N = W0.shape[1]
    return pl.pallas_call(
        _matmul_kernel,
        grid=(M // _BM,),
        in_specs=[
            pl.BlockSpec((_BM, K), lambda i: (i, 0)),
            pl.BlockSpec(memory_space=pl.ANY),
        ],
        out_specs=pl.BlockSpec((_BM, N), lambda i: (i, 0)),
        out_shape=jax.ShapeDtypeStruct((M, N), jnp.float32),
        scratch_shapes=[
            pltpu.VMEM((K, N), jnp.float32),
            pltpu.VMEM((K, N), jnp.bfloat16),
        ] + [pltpu.SemaphoreType.DMA] * _NQ,
        compiler_params=pltpu.CompilerParams(
            dimension_semantics=("arbitrary",)),
    )(x, W0)
